# Initial kernel scaffold; baseline (speedup 1.0000x reference)
import functools

import jax
import jax.numpy as jnp
from jax import lax
from jax.experimental import pallas as pl
from jax.experimental.pallas import tpu as pltpu
from jax.experimental.pallas import tpu_sc as plsc

N = 10000
E = 320000
D = 128

NUM_CORES = 2
NUM_SUBCORES = 16
NW = NUM_CORES * NUM_SUBCORES
CHUNK = 128
CHUNKS_PER_TILE = 80
E_PAD = NW * CHUNKS_PER_TILE * CHUNK
NP = 10240
ROWS_PER_SUB = NP // NUM_SUBCORES

_vector_mesh = plsc.VectorSubcoreMesh(core_axis_name="c", subcore_axis_name="s")


@functools.partial(
    pl.kernel,
    mesh=_vector_mesh,
    out_type=jax.ShapeDtypeStruct((NUM_CORES * NP, 8), jnp.float32),
    scratch_types=[
        pltpu.VMEM_SHARED((NP, 8), jnp.float32),
        pltpu.VMEM((CHUNK,), jnp.int32),
        pltpu.VMEM((CHUNK, 8), jnp.float32),
        pltpu.VMEM((ROWS_PER_SUB, 8), jnp.float32),
        pltpu.SemaphoreType.DMA,
    ],
)
def _deg_kernel(dst2_hbm, ones_hbm, zeros_hbm, degp_hbm,
                hist_sh, dstv, ones_v, zrows_v, sem):
    cid = lax.axis_index("c")
    sid = lax.axis_index("s")
    wid = cid * NUM_SUBCORES + sid

    pltpu.sync_copy(ones_hbm, ones_v)
    pltpu.sync_copy(zeros_hbm, zrows_v)
    pltpu.sync_copy(zrows_v, hist_sh.at[pl.ds(sid * ROWS_PER_SUB, ROWS_PER_SUB)])
    plsc.subcore_barrier()

    @pl.loop(0, CHUNKS_PER_TILE)
    def _(c):
        r = wid * CHUNKS_PER_TILE + c
        pltpu.sync_copy(dst2_hbm.at[r], dstv)
        pltpu.sync_copy(ones_v, hist_sh.at[dstv], add=True)

    plsc.subcore_barrier()
    pltpu.sync_copy(
        hist_sh.at[pl.ds(sid * ROWS_PER_SUB, ROWS_PER_SUB)],
        degp_hbm.at[pl.ds(cid * NP + sid * ROWS_PER_SUB, ROWS_PER_SUB)],
    )


@functools.partial(
    pl.kernel,
    mesh=_vector_mesh,
    out_type=jax.ShapeDtypeStruct((NUM_CORES * NP, D), jnp.float32),
    scratch_types=[
        pltpu.VMEM_SHARED((NP, D), jnp.float32),
        pltpu.VMEM((CHUNK,), jnp.int32),
        pltpu.VMEM((CHUNK,), jnp.int32),
        pltpu.VMEM((CHUNK,), jnp.float32),
        pltpu.VMEM((CHUNK, D), jnp.float32),
        pltpu.SemaphoreType.DMA,
    ],
)
def _agg_kernel(p_hbm, src2_hbm, dst2_hbm, w2_hbm, aggp_hbm,
                agg_sh, srcv, dstv, wv, rows_v, sem):
    cid = lax.axis_index("c")
    sid = lax.axis_index("s")
    wid = cid * NUM_SUBCORES + sid

    @pl.loop(0, CHUNK)
    def _(i):
        for j in range(D // 16):
            rows_v[i, pl.ds(j * 16, 16)] = jnp.zeros((16,), jnp.float32)

    @pl.loop(0, NP // CHUNK // NUM_SUBCORES)
    def _(k):
        z = sid + k * NUM_SUBCORES
        pltpu.sync_copy(rows_v, agg_sh.at[pl.ds(z * CHUNK, CHUNK)])

    plsc.subcore_barrier()

    @pl.loop(0, CHUNKS_PER_TILE)
    def _(c):
        r = wid * CHUNKS_PER_TILE + c
        pltpu.sync_copy(src2_hbm.at[r], srcv)
        pltpu.sync_copy(w2_hbm.at[r], wv)
        pltpu.sync_copy(dst2_hbm.at[r], dstv)
        pltpu.async_copy(p_hbm.at[srcv], rows_v, sem).wait()

        @pl.loop(0, CHUNK)
        def _(i):
            wsc = wv[i]
            for j in range(D // 16):
                sl = (i, pl.ds(j * 16, 16))
                rows_v[sl] = rows_v[sl] * wsc

        pltpu.sync_copy(rows_v, agg_sh.at[dstv], add=True)

    plsc.subcore_barrier()

    @pl.loop(0, NP // CHUNK // NUM_SUBCORES)
    def _(k):
        z = sid + k * NUM_SUBCORES
        pltpu.sync_copy(
            agg_sh.at[pl.ds(z * CHUNK, CHUNK)],
            aggp_hbm.at[pl.ds(cid * NP + z * CHUNK, CHUNK)],
        )


def _prescale_body(d0, d1, f, p):
    deg = jnp.maximum(d0[:, 0:1] + d1[:, 0:1], 1.0)
    p[...] = f[...] * lax.rsqrt(deg)


def _final_body(a0, a1, d0, d1, f, w, b, g, beta, o):
    agg = a0[...] + a1[...]
    deg = jnp.maximum(d0[:, 0:1] + d1[:, 0:1], 1.0)
    h = agg * lax.rsqrt(deg)
    h = lax.dot_general(h, w[...], (((1,), (1,)), ((), ())),
                        preferred_element_type=jnp.float32) + b[...]
    mu = jnp.mean(h, axis=1, keepdims=True)
    xc = h - mu
    var = jnp.mean(xc * xc, axis=1, keepdims=True)
    h = xc * lax.rsqrt(var + 1e-5) * g[...] + beta[...]
    h = h + f[...]
    o[...] = jnp.maximum(h, 0.0)


def kernel(feat, edge_weight, W, b, ln_gamma, ln_beta, edge_index):
    src = edge_index[0].astype(jnp.int32)
    dst = edge_index[1].astype(jnp.int32)
    w = edge_weight.astype(jnp.float32)

    pad = E_PAD - E
    pad_pos = jnp.arange(pad, dtype=jnp.int32)
    src_p = jnp.concatenate([src, pad_pos % N])
    dst_p = jnp.concatenate([dst, N + pad_pos % (NP - N)])
    w_p = jnp.concatenate([w, jnp.zeros((pad,), jnp.float32)])
    src2 = src_p.reshape(E_PAD // CHUNK, CHUNK)
    dst2 = dst_p.reshape(E_PAD // CHUNK, CHUNK)
    w2 = w_p.reshape(E_PAD // CHUNK, CHUNK)

    feat_p = jnp.pad(feat, ((0, NP - N), (0, 0)))
    ones8 = jnp.ones((CHUNK, 8), jnp.float32)
    zeros8 = jnp.zeros((ROWS_PER_SUB, 8), jnp.float32)

    degp = _deg_kernel(dst2, ones8, zeros8)

    blk = 1024
    nblk = NP // blk
    deg_spec0 = pl.BlockSpec((blk, 8), lambda i: (i, 0))
    deg_spec1 = pl.BlockSpec((blk, 8), lambda i: (nblk + i, 0))
    row_spec = pl.BlockSpec((blk, D), lambda i: (i, 0))
    p = pl.pallas_call(
        _prescale_body,
        grid=(nblk,),
        in_specs=[deg_spec0, deg_spec1, row_spec],
        out_specs=row_spec,
        out_shape=jax.ShapeDtypeStruct((NP, D), jnp.float32),
    )(degp, degp, feat_p)

    aggp = _agg_kernel(p, src2, dst2, w2)

    vec_spec = pl.BlockSpec((1, D), lambda i: (0, 0))
    out_p = pl.pallas_call(
        _final_body,
        grid=(nblk,),
        in_specs=[row_spec, pl.BlockSpec((blk, D), lambda i: (nblk + i, 0)),
                  deg_spec0, deg_spec1, row_spec,
                  pl.BlockSpec((D, D), lambda i: (0, 0)),
                  vec_spec, vec_spec, vec_spec],
        out_specs=row_spec,
        out_shape=jax.ShapeDtypeStruct((NP, D), jnp.float32),
    )(aggp, aggp, degp, degp, feat_p, W,
      b.reshape(1, D), ln_gamma.reshape(1, D), ln_beta.reshape(1, D))

    return out_p[:N]


# speedup vs baseline: 4.9677x; 4.9677x over previous
import dataclasses
import functools

import jax
import jax.numpy as jnp
from jax import lax
from jax.experimental import pallas as pl
from jax.experimental.pallas import tpu as pltpu
from jax.experimental.pallas import tpu_sc as plsc

N = 10000
E = 320000
D = 128

NUM_CORES = 2
NUM_SUBCORES = 16
NW = NUM_CORES * NUM_SUBCORES
CHUNK = 128
CHUNKS_PER_TILE = 80
E_PAD = NW * CHUNKS_PER_TILE * CHUNK
NP = 10240
NROW = NP // 128
ROWS_PER_SUB = NROW // NUM_SUBCORES


@functools.cache
def _vector_mesh():
    return plsc.VectorSubcoreMesh(core_axis_name="c", subcore_axis_name="s")


@functools.cache
def _sc_params():
    cp = pltpu.CompilerParams()
    if "needs_layout_passes" in pltpu.CompilerParams.__dataclass_fields__:
        cp = dataclasses.replace(cp, needs_layout_passes=False)
    return cp


def _deg_body(dst2_hbm, degp_hbm, hist_sh, hist_v, dstv, idbuf, sem):
    cid = lax.axis_index("c")
    sid = lax.axis_index("s")
    wid = cid * NUM_SUBCORES + sid

    @pl.loop(0, NROW)
    def _(r):
        for j in range(D // 16):
            hist_v[r, pl.ds(j * 16, 16)] = jnp.zeros((16,), jnp.float32)

    @pl.loop(0, NROW, step=16)
    def _(k):
        idbuf[pl.ds(k, 16)] = lax.iota(jnp.int32, 16) + k

    @pl.when(sid == 0)
    def _():
        pltpu.sync_copy(hist_v, hist_sh)

    plsc.subcore_barrier()

    ones16 = jnp.full((16,), 1.0, jnp.float32)

    @pl.loop(0, CHUNKS_PER_TILE)
    def _(c):
        r = wid * CHUNKS_PER_TILE + c
        pltpu.sync_copy(dst2_hbm.at[r], dstv)

        @pl.loop(0, CHUNK, step=16)
        def _(c0):
            iv = dstv[pl.ds(c0, 16)]
            plsc.addupdate_scatter(
                hist_v,
                [lax.shift_right_logical(iv, 7), lax.bitwise_and(iv, 127)],
                ones16)

    pltpu.sync_copy(hist_v, hist_sh.at[idbuf], add=True)
    plsc.subcore_barrier()

    @pl.when(sid < NROW // 8)
    def _():
        pltpu.sync_copy(
            hist_sh.at[pl.ds(sid * 8, 8)],
            degp_hbm.at[pl.ds(cid * NROW + sid * 8, 8)],
        )


def _agg_body(feat_hbm, src2_hbm, dst2_hbm, w2_hbm, norm_hbm, aggp_hbm,
              agg_sh, norm_v, srcv, dstv, wv, rows_v, sem):
    cid = lax.axis_index("c")
    sid = lax.axis_index("s")
    wid = cid * NUM_SUBCORES + sid

    pltpu.sync_copy(norm_hbm, norm_v)

    @pl.loop(0, CHUNK)
    def _(i):
        for j in range(D // 16):
            rows_v[i, pl.ds(j * 16, 16)] = jnp.zeros((16,), jnp.float32)

    @pl.loop(0, ROWS_PER_SUB)
    def _(k):
        z = sid + k * NUM_SUBCORES
        pltpu.sync_copy(rows_v, agg_sh.at[pl.ds(z * CHUNK, CHUNK)])

    plsc.subcore_barrier()

    @pl.loop(0, CHUNKS_PER_TILE)
    def _(c):
        r = wid * CHUNKS_PER_TILE + c
        pltpu.sync_copy(src2_hbm.at[r], srcv)
        pltpu.sync_copy(w2_hbm.at[r], wv)
        pltpu.sync_copy(dst2_hbm.at[r], dstv)
        pltpu.async_copy(feat_hbm.at[srcv], rows_v, sem).wait()

        @pl.loop(0, CHUNK, step=16)
        def _(c0):
            iv = srcv[pl.ds(c0, 16)]
            nsrc = plsc.load_gather(
                norm_v,
                [lax.shift_right_logical(iv, 7), lax.bitwise_and(iv, 127)])
            cw = wv[pl.ds(c0, 16)] * nsrc
            for l in range(16):
                wsc = cw[l]
                for j in range(D // 16):
                    sl = (c0 + l, pl.ds(j * 16, 16))
                    rows_v[sl] = rows_v[sl] * wsc

        pltpu.sync_copy(rows_v, agg_sh.at[dstv], add=True)

    plsc.subcore_barrier()

    @pl.loop(0, ROWS_PER_SUB)
    def _(k):
        z = sid + k * NUM_SUBCORES
        pltpu.sync_copy(agg_sh.at[pl.ds(z * CHUNK, CHUNK)], rows_v)

        @pl.loop(0, CHUNK, step=16)
        def _(c0):
            nv = norm_v[z, pl.ds(c0, 16)]
            for l in range(16):
                wsc = nv[l]
                for j in range(D // 16):
                    sl = (c0 + l, pl.ds(j * 16, 16))
                    rows_v[sl] = rows_v[sl] * wsc

        pltpu.sync_copy(rows_v, aggp_hbm.at[pl.ds(cid * NP + z * CHUNK, CHUNK)])


def _norm_body(dref, nref):
    deg = jnp.maximum(dref[:NROW] + dref[NROW:], 1.0)
    nref[...] = lax.rsqrt(deg)


def _final_body(a0, a1, f, w, b, g, beta, o):
    h = a0[...] + a1[...]
    h = lax.dot_general(h, w[...], (((1,), (1,)), ((), ())),
                        preferred_element_type=jnp.float32) + b[...]
    mu = jnp.mean(h, axis=1, keepdims=True)
    xc = h - mu
    var = jnp.mean(xc * xc, axis=1, keepdims=True)
    h = xc * lax.rsqrt(var + 1e-5) * g[...] + beta[...]
    h = h + f[...]
    o[...] = jnp.maximum(h, 0.0)


def kernel(feat, edge_weight, W, b, ln_gamma, ln_beta, edge_index):
    src = edge_index[0].astype(jnp.int32)
    dst = edge_index[1].astype(jnp.int32)
    w = edge_weight.astype(jnp.float32)

    pad = E_PAD - E
    pad_pos = jnp.arange(pad, dtype=jnp.int32)
    src_p = jnp.concatenate([src, pad_pos % N])
    dst_p = jnp.concatenate([dst, N + pad_pos % (NP - N)])
    w_p = jnp.concatenate([w, jnp.zeros((pad,), jnp.float32)])
    src2 = src_p.reshape(E_PAD // CHUNK, CHUNK)
    dst2 = dst_p.reshape(E_PAD // CHUNK, CHUNK)
    w2 = w_p.reshape(E_PAD // CHUNK, CHUNK)

    feat_p = jnp.pad(feat, ((0, NP - N), (0, 0)))

    deg_kernel = pl.kernel(
        _deg_body,
        mesh=_vector_mesh(),
        compiler_params=_sc_params(),
        out_type=jax.ShapeDtypeStruct((NUM_CORES * NROW, D), jnp.float32),
        scratch_types=[
            pltpu.VMEM_SHARED((NROW, D), jnp.float32),
            pltpu.VMEM((NROW, D), jnp.float32),
            pltpu.VMEM((CHUNK,), jnp.int32),
            pltpu.VMEM((NROW,), jnp.int32),
            pltpu.SemaphoreType.DMA,
        ],
    )
    degp = deg_kernel(dst2)

    norm2d = pl.pallas_call(
        _norm_body,
        out_shape=jax.ShapeDtypeStruct((NROW, D), jnp.float32),
    )(degp)

    agg_kernel = pl.kernel(
        _agg_body,
        mesh=_vector_mesh(),
        compiler_params=_sc_params(),
        out_type=jax.ShapeDtypeStruct((NUM_CORES * NP, D), jnp.float32),
        scratch_types=[
            pltpu.VMEM_SHARED((NP, D), jnp.float32),
            pltpu.VMEM((NROW, D), jnp.float32),
            pltpu.VMEM((CHUNK,), jnp.int32),
            pltpu.VMEM((CHUNK,), jnp.int32),
            pltpu.VMEM((CHUNK,), jnp.float32),
            pltpu.VMEM((CHUNK, D), jnp.float32),
            pltpu.SemaphoreType.DMA,
        ],
    )
    aggp = agg_kernel(feat_p, src2, dst2, w2, norm2d)

    blk = 1024
    nblk = NP // blk
    row_spec = pl.BlockSpec((blk, D), lambda i: (i, 0))
    vec_spec = pl.BlockSpec((1, D), lambda i: (0, 0))
    out_p = pl.pallas_call(
        _final_body,
        grid=(nblk,),
        in_specs=[row_spec, pl.BlockSpec((blk, D), lambda i: (nblk + i, 0)),
                  row_spec,
                  pl.BlockSpec((D, D), lambda i: (0, 0)),
                  vec_spec, vec_spec, vec_spec],
        out_specs=row_spec,
        out_shape=jax.ShapeDtypeStruct((NP, D), jnp.float32),
    )(aggp, aggp, feat_p, W,
      b.reshape(1, D), ln_gamma.reshape(1, D), ln_beta.reshape(1, D))

    return out_p[:N]


# speedup vs baseline: 9.7489x; 1.9625x over previous
import dataclasses
import functools

import jax
import jax.numpy as jnp
from jax import lax
from jax.experimental import pallas as pl
from jax.experimental.pallas import tpu as pltpu
from jax.experimental.pallas import tpu_sc as plsc

N = 10000
E = 320000
D = 128

NUM_CORES = 2
NUM_SUBCORES = 16
NW = NUM_CORES * NUM_SUBCORES
CHUNK = 128
CPT = 80
E_PAD = NW * CPT * CHUNK
NCHUNKS = E_PAD // CHUNK
HCH = NCHUNKS // NUM_SUBCORES
NP = 10240
NROW = NP // 128
MB = 4
NMB = CPT // MB


@functools.cache
def _vector_mesh():
    return plsc.VectorSubcoreMesh(core_axis_name="c", subcore_axis_name="s")


@functools.cache
def _sc_params():
    cp = pltpu.CompilerParams()
    if "needs_layout_passes" in pltpu.CompilerParams.__dataclass_fields__:
        cp = dataclasses.replace(cp, needs_layout_passes=False)
    return cp


def _rsqrt16(x):
    i = plsc.bitcast(x, jnp.int32)
    i = jnp.int32(0x5F3759DF) - lax.shift_right_arithmetic(i, 1)
    y = plsc.bitcast(i, jnp.float32)
    for _ in range(3):
        y = y * (1.5 - 0.5 * x * y * y)
    return y


def _scale_rows(buf, c0, cw):
    for l in range(16):
        wsc = cw[l]
        for j in range(D // 16):
            sl = (c0 + l, pl.ds(j * 16, 16))
            buf[sl] = buf[sl] * wsc


def _sc_body(feat_hbm, meta_hbm, aggp_hbm,
             agg_sh, hist_sh, norm_v, meta_a, meta_b, idbuf,
             rows0, rows1, gsem0, gsem1, ssem0, ssem1, msem):
    cid = lax.axis_index("c")
    sid = lax.axis_index("s")
    wid = cid * NUM_SUBCORES + sid
    rows = (rows0, rows1)
    gsems = (gsem0, gsem1)
    ssems = (ssem0, ssem1)
    metas = (meta_a, meta_b)

    def drain(buf, sem):
        pltpu.make_async_copy(feat_hbm.at[pl.ds(0, CHUNK)], buf, sem).wait()

    @pl.loop(0, NROW)
    def _(r):
        for j in range(D // 16):
            norm_v[r, pl.ds(j * 16, 16)] = jnp.zeros((16,), jnp.float32)

    @pl.loop(0, CHUNK)
    def _(i):
        for j in range(D // 16):
            rows0[i, pl.ds(j * 16, 16)] = jnp.zeros((16,), jnp.float32)

    @pl.loop(0, NROW // NUM_SUBCORES)
    def _(k):
        z = sid + k * NUM_SUBCORES
        pltpu.sync_copy(rows0, agg_sh.at[pl.ds(z * CHUNK, CHUNK)])

    @pl.when(sid == 0)
    def _():
        pltpu.sync_copy(norm_v, hist_sh)

    @pl.loop(0, NROW, step=16)
    def _(k):
        idbuf[pl.ds(k, 16)] = lax.iota(jnp.int32, 16) + k

    plsc.subcore_barrier()

    ones16 = jnp.full((16,), 1.0, jnp.float32)
    hbase = sid * HCH * 4

    def _hist_block(mref):
        for cc in range(MB):
            @pl.loop(0, CHUNK, step=16)
            def _(c0):
                iv = mref[4 * cc + 1, pl.ds(c0, 16)]
                plsc.addupdate_scatter(
                    norm_v,
                    [lax.shift_right_logical(iv, 7), lax.bitwise_and(iv, 127)],
                    ones16)

    def _mdrain(mref):
        pltpu.make_async_copy(meta_hbm.at[pl.ds(0, MB * 4)], mref, msem).wait()

    pltpu.async_copy(meta_hbm.at[pl.ds(hbase, MB * 4)], meta_a, msem)

    @pl.loop(0, HCH // MB // 2)
    def _(u):
        _mdrain(meta_a)
        pltpu.async_copy(meta_hbm.at[pl.ds(hbase + (2 * u + 1) * MB * 4, MB * 4)],
                         meta_b, msem)
        _hist_block(meta_a)
        _mdrain(meta_b)

        @pl.when(u + 1 < HCH // MB // 2)
        def _():
            pltpu.async_copy(meta_hbm.at[pl.ds(hbase + (2 * u + 2) * MB * 4, MB * 4)],
                             meta_a, msem)

        _hist_block(meta_b)

    pltpu.sync_copy(norm_v, hist_sh.at[idbuf], add=True)
    plsc.subcore_barrier()

    pltpu.sync_copy(hist_sh, norm_v)

    @pl.loop(0, NROW)
    def _(r):
        for j in range(D // 16):
            sl = (r, pl.ds(j * 16, 16))
            norm_v[sl] = _rsqrt16(jnp.maximum(norm_v[sl], 1.0))

    base = wid * CPT * 4
    pltpu.sync_copy(meta_hbm.at[pl.ds(base, MB * 4)], meta_a)
    pltpu.async_copy(feat_hbm.at[meta_a.at[0]], rows0, gsem0)
    pltpu.async_copy(feat_hbm.at[meta_a.at[4]], rows1, gsem1)

    @pl.loop(0, CPT // (2 * MB))
    def _(t):
        for j in range(2 * MB):
            cc = j % MB
            b = rows[j % 2]
            mref = metas[j // MB]
            drain(b, gsems[j % 2])

            if j == 0:
                pltpu.async_copy(
                    meta_hbm.at[pl.ds(base + (2 * t + 1) * MB * 4, MB * 4)],
                    meta_b, msem)
            if j == 2:
                _mdrain(meta_b)
            if j == MB:
                @pl.when(t + 1 < CPT // (2 * MB))
                def _():
                    pltpu.async_copy(
                        meta_hbm.at[pl.ds(base + (2 * t + 2) * MB * 4, MB * 4)],
                        meta_a, msem)
            if j == MB + 2:
                @pl.when(t + 1 < CPT // (2 * MB))
                def _():
                    _mdrain(meta_a)

            @pl.loop(0, CHUNK, step=16)
            def _(c0):
                iv = mref[4 * cc, pl.ds(c0, 16)]
                nsrc = plsc.load_gather(
                    norm_v,
                    [lax.shift_right_logical(iv, 7), lax.bitwise_and(iv, 127)])
                wv = plsc.bitcast(mref[4 * cc + 2, pl.ds(c0, 16)], jnp.float32)
                _scale_rows(b, c0, wv * nsrc)

            pltpu.async_copy(b, agg_sh.at[mref.at[4 * cc + 1]], ssems[j % 2],
                             add=True)
            drain(b, ssems[j % 2])

            j2 = j + 2
            if j2 < 2 * MB:
                pltpu.async_copy(feat_hbm.at[metas[j2 // MB].at[4 * (j2 % MB)]],
                                 b, gsems[j % 2])
            else:
                @pl.when(t + 1 < CPT // (2 * MB))
                def _():
                    pltpu.async_copy(feat_hbm.at[meta_a.at[4 * (j2 % MB)]],
                                     b, gsems[j % 2])

    plsc.subcore_barrier()

    @pl.loop(0, NROW // NUM_SUBCORES)
    def _(k):
        z = sid + k * NUM_SUBCORES
        pltpu.sync_copy(agg_sh.at[pl.ds(z * CHUNK, CHUNK)], rows0)

        @pl.loop(0, CHUNK, step=16)
        def _(c0):
            _scale_rows(rows0, c0, norm_v[z, pl.ds(c0, 16)])

        pltpu.sync_copy(rows0, aggp_hbm.at[pl.ds(cid * NP + z * CHUNK, CHUNK)])


def _final_body(a0, a1, f, w, b, g, beta, o):
    h = a0[...] + a1[...]
    h = lax.dot_general(h, w[...], (((1,), (1,)), ((), ())),
                        preferred_element_type=jnp.float32) + b[...]
    mu = jnp.mean(h, axis=1, keepdims=True)
    xc = h - mu
    var = jnp.mean(xc * xc, axis=1, keepdims=True)
    h = xc * lax.rsqrt(var + 1e-5) * g[...] + beta[...]
    h = h + f[...]
    o[...] = jnp.maximum(h, 0.0)


def kernel(feat, edge_weight, W, b, ln_gamma, ln_beta, edge_index):
    src = edge_index[0].astype(jnp.int32)
    dst = edge_index[1].astype(jnp.int32)
    w = edge_weight.astype(jnp.float32)

    pad = E_PAD - E
    pad_pos = jnp.arange(pad, dtype=jnp.int32)
    src_p = jnp.concatenate([src, pad_pos % N])
    dst_p = jnp.concatenate([dst, N + pad_pos % (NP - N)])
    w_p = jnp.concatenate([w, jnp.zeros((pad,), jnp.float32)])
    meta = jnp.stack(
        [src_p.reshape(NCHUNKS, CHUNK),
         dst_p.reshape(NCHUNKS, CHUNK),
         lax.bitcast_convert_type(w_p, jnp.int32).reshape(NCHUNKS, CHUNK)],
        axis=1)
    meta = jnp.concatenate(
        [meta, jnp.zeros((NCHUNKS, 1, CHUNK), jnp.int32)],
        axis=1).reshape(NCHUNKS * 4, CHUNK)

    feat_p = jnp.pad(feat, ((0, NP - N), (0, 0)))

    sc_kernel = pl.kernel(
        _sc_body,
        mesh=_vector_mesh(),
        compiler_params=_sc_params(),
        out_type=jax.ShapeDtypeStruct((NUM_CORES * NP, D), jnp.float32),
        scratch_types=[
            pltpu.VMEM_SHARED((NP, D), jnp.float32),
            pltpu.VMEM_SHARED((NROW, D), jnp.float32),
            pltpu.VMEM((NROW, D), jnp.float32),
            pltpu.VMEM((MB * 4, CHUNK), jnp.int32),
            pltpu.VMEM((MB * 4, CHUNK), jnp.int32),
            pltpu.VMEM((NROW,), jnp.int32),
            pltpu.VMEM((CHUNK, D), jnp.float32),
            pltpu.VMEM((CHUNK, D), jnp.float32),
            pltpu.SemaphoreType.DMA,
            pltpu.SemaphoreType.DMA,
            pltpu.SemaphoreType.DMA,
            pltpu.SemaphoreType.DMA,
            pltpu.SemaphoreType.DMA,
        ],
    )
    aggp = sc_kernel(feat_p, meta)

    blk = 1024
    nblk = NP // blk
    row_spec = pl.BlockSpec((blk, D), lambda i: (i, 0))
    vec_spec = pl.BlockSpec((1, D), lambda i: (0, 0))
    out_p = pl.pallas_call(
        _final_body,
        grid=(nblk,),
        in_specs=[row_spec, pl.BlockSpec((blk, D), lambda i: (nblk + i, 0)),
                  row_spec,
                  pl.BlockSpec((D, D), lambda i: (0, 0)),
                  vec_spec, vec_spec, vec_spec],
        out_specs=row_spec,
        out_shape=jax.ShapeDtypeStruct((NP, D), jnp.float32),
    )(aggp, aggp, feat_p, W,
      b.reshape(1, D), ln_gamma.reshape(1, D), ln_beta.reshape(1, D))

    return out_p[:N]


# speedup vs baseline: 10.6956x; 1.0971x over previous
import dataclasses
import functools

import jax
import jax.numpy as jnp
from jax import lax
from jax.experimental import pallas as pl
from jax.experimental.pallas import tpu as pltpu
from jax.experimental.pallas import tpu_sc as plsc

N = 10000
E = 320000
D = 128

NUM_CORES = 2
NUM_SUBCORES = 16
NW = NUM_CORES * NUM_SUBCORES
CHUNK = 128
CPT = 80
E_PAD = NW * CPT * CHUNK
NCHUNKS = E_PAD // CHUNK
HCH = NCHUNKS // NUM_SUBCORES
NP = 10240
NROW = NP // 128
MB = 4
NMB = CPT // MB


@functools.cache
def _vector_mesh():
    return plsc.VectorSubcoreMesh(core_axis_name="c", subcore_axis_name="s")


@functools.cache
def _sc_params():
    cp = pltpu.CompilerParams()
    if "needs_layout_passes" in pltpu.CompilerParams.__dataclass_fields__:
        cp = dataclasses.replace(cp, needs_layout_passes=False)
    return cp


def _rsqrt16(x):
    i = plsc.bitcast(x, jnp.int32)
    i = jnp.int32(0x5F3759DF) - lax.shift_right_arithmetic(i, 1)
    y = plsc.bitcast(i, jnp.float32)
    for _ in range(3):
        y = y * (1.5 - 0.5 * x * y * y)
    return y


def _scale_rows(buf, c0, cw):
    for l in range(16):
        wsc = cw[l]
        for j in range(D // 16):
            sl = (c0 + l, pl.ds(j * 16, 16))
            buf[sl] = buf[sl] * wsc


def _sc_body(feat_hbm, meta_hbm, dst2_hbm, aggp_hbm,
             agg_sh, hist_sh, norm_v, meta_a, meta_b, idbuf,
             rows0, rows1, gsem0, gsem1, ssem0, ssem1, msem):
    cid = lax.axis_index("c")
    sid = lax.axis_index("s")
    wid = cid * NUM_SUBCORES + sid
    rows = (rows0, rows1)
    gsems = (gsem0, gsem1)
    ssems = (ssem0, ssem1)
    metas = (meta_a, meta_b)

    def drain(buf, sem):
        pltpu.make_async_copy(feat_hbm.at[pl.ds(0, CHUNK)], buf, sem).wait()

    @pl.loop(0, NROW)
    def _(r):
        for j in range(D // 16):
            norm_v[r, pl.ds(j * 16, 16)] = jnp.zeros((16,), jnp.float32)

    @pl.loop(0, CHUNK)
    def _(i):
        for j in range(D // 16):
            rows0[i, pl.ds(j * 16, 16)] = jnp.zeros((16,), jnp.float32)

    for k in range(NROW // NUM_SUBCORES):
        z = sid + k * NUM_SUBCORES
        pltpu.async_copy(rows0, agg_sh.at[pl.ds(z * CHUNK, CHUNK)], ssem0)
    for k in range(NROW // NUM_SUBCORES):
        drain(rows0, ssem0)

    @pl.when(sid == 0)
    def _():
        pltpu.sync_copy(norm_v, hist_sh)

    @pl.loop(0, NROW, step=16)
    def _(k):
        idbuf[pl.ds(k, 16)] = lax.iota(jnp.int32, 16) + k

    plsc.subcore_barrier()

    ones16 = jnp.full((16,), 1.0, jnp.float32)
    HB = MB * 4
    hbase = sid * HCH

    def _hist_block(mref):
        for cc in range(HB):
            @pl.loop(0, CHUNK, step=16)
            def _(c0):
                iv = mref[cc, pl.ds(c0, 16)]
                plsc.addupdate_scatter(
                    norm_v,
                    [lax.shift_right_logical(iv, 7), lax.bitwise_and(iv, 127)],
                    ones16)

    def _mdrain(mref):
        pltpu.make_async_copy(dst2_hbm.at[pl.ds(0, HB)], mref, msem).wait()

    pltpu.async_copy(dst2_hbm.at[pl.ds(hbase, HB)], meta_a, msem)

    @pl.loop(0, HCH // HB // 2)
    def _(u):
        _mdrain(meta_a)
        pltpu.async_copy(dst2_hbm.at[pl.ds(hbase + (2 * u + 1) * HB, HB)],
                         meta_b, msem)
        _hist_block(meta_a)
        _mdrain(meta_b)

        @pl.when(u + 1 < HCH // HB // 2)
        def _():
            pltpu.async_copy(dst2_hbm.at[pl.ds(hbase + (2 * u + 2) * HB, HB)],
                             meta_a, msem)

        _hist_block(meta_b)

    pltpu.sync_copy(norm_v, hist_sh.at[idbuf], add=True)
    plsc.subcore_barrier()

    base = wid * CPT * 4
    pltpu.sync_copy(meta_hbm.at[pl.ds(base, MB * 4)], meta_a)
    pltpu.async_copy(feat_hbm.at[meta_a.at[0]], rows0, gsem0)
    pltpu.async_copy(feat_hbm.at[meta_a.at[4]], rows1, gsem1)

    pltpu.sync_copy(hist_sh, norm_v)

    @pl.loop(0, NROW)
    def _(r):
        for j in range(D // 16):
            sl = (r, pl.ds(j * 16, 16))
            norm_v[sl] = _rsqrt16(jnp.maximum(norm_v[sl], 1.0))

    @pl.loop(0, CPT // (2 * MB))
    def _(t):
        for j in range(2 * MB):
            cc = j % MB
            b = rows[j % 2]
            mref = metas[j // MB]
            drain(b, gsems[j % 2])

            if j == 0:
                pltpu.async_copy(
                    meta_hbm.at[pl.ds(base + (2 * t + 1) * MB * 4, MB * 4)],
                    meta_b, msem)
            if j == 2:
                _mdrain(meta_b)
            if j == MB:
                @pl.when(t + 1 < CPT // (2 * MB))
                def _():
                    pltpu.async_copy(
                        meta_hbm.at[pl.ds(base + (2 * t + 2) * MB * 4, MB * 4)],
                        meta_a, msem)
            if j == MB + 2:
                @pl.when(t + 1 < CPT // (2 * MB))
                def _():
                    _mdrain(meta_a)

            @pl.loop(0, CHUNK, step=16)
            def _(c0):
                iv = mref[4 * cc, pl.ds(c0, 16)]
                nsrc = plsc.load_gather(
                    norm_v,
                    [lax.shift_right_logical(iv, 7), lax.bitwise_and(iv, 127)])
                wv = plsc.bitcast(mref[4 * cc + 2, pl.ds(c0, 16)], jnp.float32)
                _scale_rows(b, c0, wv * nsrc)

            pltpu.async_copy(b, agg_sh.at[mref.at[4 * cc + 1]], ssems[j % 2],
                             add=True)
            drain(b, ssems[j % 2])

            j2 = j + 2
            if j2 < 2 * MB:
                pltpu.async_copy(feat_hbm.at[metas[j2 // MB].at[4 * (j2 % MB)]],
                                 b, gsems[j % 2])
            else:
                @pl.when(t + 1 < CPT // (2 * MB))
                def _():
                    pltpu.async_copy(feat_hbm.at[meta_a.at[4 * (j2 % MB)]],
                                     b, gsems[j % 2])

    plsc.subcore_barrier()

    for k in range(NROW // NUM_SUBCORES):
        z = sid + k * NUM_SUBCORES
        buf = rows[k % 2]
        if k >= 2:
            drain(buf, ssems[k % 2])
        pltpu.sync_copy(agg_sh.at[pl.ds(z * CHUNK, CHUNK)], buf)

        @pl.loop(0, CHUNK, step=16)
        def _(c0):
            _scale_rows(buf, c0, norm_v[z, pl.ds(c0, 16)])

        pltpu.async_copy(buf, aggp_hbm.at[pl.ds(cid * NP + z * CHUNK, CHUNK)],
                         ssems[k % 2])
    drain(rows1, ssem1)
    drain(rows0, ssem0)


def _final_body(a0, a1, f, w, b, g, beta, o):
    h = a0[...] + a1[...]
    h = lax.dot_general(h, w[...], (((1,), (1,)), ((), ())),
                        preferred_element_type=jnp.float32) + b[...]
    mu = jnp.mean(h, axis=1, keepdims=True)
    xc = h - mu
    var = jnp.mean(xc * xc, axis=1, keepdims=True)
    h = xc * lax.rsqrt(var + 1e-5) * g[...] + beta[...]
    h = h + f[...]
    o[...] = jnp.maximum(h, 0.0)


def kernel(feat, edge_weight, W, b, ln_gamma, ln_beta, edge_index):
    src = edge_index[0].astype(jnp.int32)
    dst = edge_index[1].astype(jnp.int32)
    w = edge_weight.astype(jnp.float32)

    pad = E_PAD - E
    pad_pos = jnp.arange(pad, dtype=jnp.int32)
    src_p = jnp.concatenate([src, pad_pos % N])
    dst_p = jnp.concatenate([dst, N + pad_pos % (NP - N)])
    w_p = jnp.concatenate([w, jnp.zeros((pad,), jnp.float32)])
    meta = jnp.stack(
        [src_p.reshape(NCHUNKS, CHUNK),
         dst_p.reshape(NCHUNKS, CHUNK),
         lax.bitcast_convert_type(w_p, jnp.int32).reshape(NCHUNKS, CHUNK)],
        axis=1)
    meta = jnp.concatenate(
        [meta, jnp.zeros((NCHUNKS, 1, CHUNK), jnp.int32)],
        axis=1).reshape(NCHUNKS * 4, CHUNK)

    feat_p = jnp.pad(feat, ((0, NP - N), (0, 0)))

    sc_kernel = pl.kernel(
        _sc_body,
        mesh=_vector_mesh(),
        compiler_params=_sc_params(),
        out_type=jax.ShapeDtypeStruct((NUM_CORES * NP, D), jnp.float32),
        scratch_types=[
            pltpu.VMEM_SHARED((NP, D), jnp.float32),
            pltpu.VMEM_SHARED((NROW, D), jnp.float32),
            pltpu.VMEM((NROW, D), jnp.float32),
            pltpu.VMEM((MB * 4, CHUNK), jnp.int32),
            pltpu.VMEM((MB * 4, CHUNK), jnp.int32),
            pltpu.VMEM((NROW,), jnp.int32),
            pltpu.VMEM((CHUNK, D), jnp.float32),
            pltpu.VMEM((CHUNK, D), jnp.float32),
            pltpu.SemaphoreType.DMA,
            pltpu.SemaphoreType.DMA,
            pltpu.SemaphoreType.DMA,
            pltpu.SemaphoreType.DMA,
            pltpu.SemaphoreType.DMA,
        ],
    )
    aggp = sc_kernel(feat_p, meta, dst_p.reshape(NCHUNKS, CHUNK))

    blk = 1024
    nblk = NP // blk
    row_spec = pl.BlockSpec((blk, D), lambda i: (i, 0))
    vec_spec = pl.BlockSpec((1, D), lambda i: (0, 0))
    out_p = pl.pallas_call(
        _final_body,
        grid=(nblk,),
        in_specs=[row_spec, pl.BlockSpec((blk, D), lambda i: (nblk + i, 0)),
                  row_spec,
                  pl.BlockSpec((D, D), lambda i: (0, 0)),
                  vec_spec, vec_spec, vec_spec],
        out_specs=row_spec,
        out_shape=jax.ShapeDtypeStruct((NP, D), jnp.float32),
    )(aggp, aggp, feat_p, W,
      b.reshape(1, D), ln_gamma.reshape(1, D), ln_beta.reshape(1, D))

    return out_p[:N]
